# baseline (speedup 1.0000x reference)
import functools

import jax
import jax.numpy as jnp
from jax import lax
from jax.experimental import pallas as pl
from jax.experimental.pallas import tpu as pltpu
from jax.experimental.pallas import tpu_sc as plsc


@functools.lru_cache(maxsize=None)
def _build_lookup(V, D, B, N1):
    info = plsc.get_sparse_core_info()
    NC, NS, L = info.num_cores, info.num_subcores, info.num_lanes
    NW = NC * NS
    assert B % NW == 0 and D % L == 0
    per_w = B // NW
    GR = L
    n_g = per_w // GR
    assert n_g % 2 == 0
    mesh = plsc.VectorSubcoreMesh(core_axis_name="c", subcore_axis_name="s")

    @functools.partial(
        pl.kernel,
        mesh=mesh,
        out_type=jax.ShapeDtypeStruct((B, D), jnp.int32),
        scratch_types=[
            pltpu.VMEM((per_w,), jnp.int32),
            pltpu.VMEM((GR, D), jnp.float32),
            pltpu.VMEM((GR, D), jnp.float32),
            pltpu.VMEM((GR, D), jnp.int32),
            pltpu.VMEM((GR, D), jnp.int32),
            pltpu.SemaphoreType.DMA,
            pltpu.SemaphoreType.DMA,
            pltpu.SemaphoreType.DMA,
            pltpu.SemaphoreType.DMA,
        ],
    )
    def lookup(x_hbm, ow_hbm, nw_hbm, out_hbm,
               idx_v, buf0, buf1, outb0, outb1,
               gsem0, gsem1, osem0, osem1):
        wid = lax.axis_index("s") * NC + lax.axis_index("c")
        base = wid * per_w
        pltpu.sync_copy(x_hbm.at[pl.ds(base, per_w)], idx_v)

        def issue(g, buf, gsem):
            ivec = idx_v[pl.ds(g * GR, GR)]
            for r in range(GR):
                iv = ivec[r]
                good = iv < V

                @pl.when(good)
                def _():
                    pltpu.async_copy(ow_hbm.at[iv], buf.at[r], gsem)

                @pl.when(jnp.logical_not(good))
                def _():
                    pltpu.async_copy(nw_hbm.at[iv - (V - 1)], buf.at[r], gsem)

        def wait_rows(buf, gsem):
            pltpu.make_async_copy(ow_hbm.at[pl.ds(0, GR)], buf, gsem).wait()

        def convert(buf, outb):
            for r in range(GR):
                for c in range(D // L):
                    cs = pl.ds(c * L, L)
                    outb[r, cs] = buf[r, cs].astype(jnp.int32)

        def half(i, g, buf, outb, gsem, osem):
            wait_rows(buf, gsem)

            @pl.when(i >= 1)
            def _():
                pltpu.make_async_copy(
                    outb, out_hbm.at[pl.ds(0, GR)], osem).wait()

            convert(buf, outb)
            pltpu.async_copy(outb, out_hbm.at[pl.ds(base + g * GR, GR)], osem)

            @pl.when(g + 2 < n_g)
            def _():
                issue(g + 2, buf, gsem)

        issue(0, buf0, gsem0)
        issue(1, buf1, gsem1)

        def pair_body(i, _):
            half(i, 2 * i, buf0, outb0, gsem0, osem0)
            half(i, 2 * i + 1, buf1, outb1, gsem1, osem1)
            return 0

        lax.fori_loop(0, n_g // 2, pair_body, 0)
        pltpu.make_async_copy(outb0, out_hbm.at[pl.ds(0, GR)], osem0).wait()
        pltpu.make_async_copy(outb1, out_hbm.at[pl.ds(0, GR)], osem1).wait()

    return lookup


def kernel(x, original_weight, new_weight):
    V, D = original_weight.shape
    N1 = new_weight.shape[0]
    Bt, S = x.shape
    B = Bt * S
    lookup = _build_lookup(V, D, B, N1)
    out = lookup(x.reshape(B), original_weight, new_weight)
    return out.reshape(Bt, S, D)
